# baseline (speedup 1.0000x reference)
import functools

import jax
import jax.numpy as jnp
from jax import lax
from jax.experimental import pallas as pl
from jax.experimental.pallas import tpu as pltpu
from jax.experimental.pallas import tpu_sc as plsc

B, C, H, W, K = 32, 2, 128, 128, 128
HW = H * W
L = 16
ROWS_PER_MAP = HW // L
NS = 16
NB = B // NS


def _body(tab_hbm, pk_hbm, out_hbm,
          pk_v, g0_v, g1_v, acc2_v, all_v, loss_v, shared,
          sem_pk, sem_g0, sem_g1):
    s = lax.axis_index("s")
    iot = lax.iota(jnp.int32, L)

    cp_pk = [pltpu.async_copy(pk_hbm.at[s * NB + i], pk_v.at[i], sem_pk)
             for i in range(NB)]
    cp_g = []
    for i in range(NB):
        cp_pk[i].wait()
        sem_g = sem_g0 if i == 0 else sem_g1
        cp_g.append((pltpu.async_copy(tab_hbm.at[pk_v.at[i, 0]], g0_v.at[i], sem_g),
                     pltpu.async_copy(tab_hbm.at[pk_v.at[i, 1]], g1_v.at[i], sem_g)))
    acc = jnp.zeros((L,), jnp.float32)
    macc = jnp.zeros((L,), jnp.float32)
    for i in range(NB):
        cp_g[i][0].wait()
        cp_g[i][1].wait()

        def chunk(j, carry, i=i):
            a, ma = carry
            sl = pl.ds(j * L, L)
            rem = pk_v[i, 2, sl]
            m = plsc.bitcast(pk_v[i, 3, sl], jnp.float32)
            t0 = plsc.bitcast(pk_v[i, 4, sl], jnp.float32)
            t1 = plsc.bitcast(pk_v[i, 5, sl], jnp.float32)
            kk = iot + j * L
            p0 = plsc.load_gather(g0_v, [jnp.full((L,), i), kk, rem])
            p1 = plsc.load_gather(g1_v, [jnp.full((L,), i), kk, rem])
            a = (a
                 + jnp.abs(p0 / (t0 + 1e-4) * m - m)
                 + jnp.abs(p1 / (t1 + 1e-4) * m - m))
            return a, ma + m + m

        acc, macc = lax.fori_loop(0, K // L, chunk, (acc, macc))
    acc2_v[0, :] = acc
    acc2_v[1, :] = macc
    pltpu.sync_copy(acc2_v, shared.at[pl.ds(2 * s, 2)])
    plsc.subcore_barrier()

    @pl.when(s == 0)
    def _reduce():
        pltpu.sync_copy(shared, all_v)

        def acc_row(j, carry):
            a, mm = carry
            return a + all_v[2 * j], mm + all_v[2 * j + 1]

        a, mm = lax.fori_loop(
            0, NS, acc_row,
            (jnp.zeros((L,), jnp.float32), jnp.zeros((L,), jnp.float32)))
        total = jnp.full((L,), jnp.sum(a))
        mtotal = jnp.full((L,), jnp.sum(mm))
        loss_v[...] = total / (mtotal + 1e-4)
        pltpu.sync_copy(loss_v.at[pl.ds(0, 8)], out_hbm)


@jax.jit
def kernel(output, mask, ind, target):
    tab = output.reshape(B * C * ROWS_PER_MAP, L)
    ind32 = ind.astype(jnp.int32)
    plane = jnp.arange(B, dtype=jnp.int32)[:, None] * (C * ROWS_PER_MAP)
    row0 = (ind32 >> 4) + plane
    row1 = row0 + ROWS_PER_MAP
    rem = ind32 & 15
    m32 = lax.bitcast_convert_type(mask, jnp.int32)
    t32 = lax.bitcast_convert_type(jnp.transpose(target, (0, 2, 1)), jnp.int32)
    packed = jnp.concatenate(
        [row0[:, None], row1[:, None], rem[:, None], m32[:, None], t32],
        axis=1)
    mesh = plsc.VectorSubcoreMesh(
        core_axis_name="c", subcore_axis_name="s", num_cores=1)
    run = functools.partial(
        pl.kernel,
        mesh=mesh,
        compiler_params=pltpu.CompilerParams(
            needs_layout_passes=False, use_tc_tiling_on_sc=False),
        out_type=jax.ShapeDtypeStruct((8,), jnp.float32),
        scratch_types=[
            pltpu.VMEM((NB, 6, K), jnp.int32),
            pltpu.VMEM((NB, K, L), jnp.float32),
            pltpu.VMEM((NB, K, L), jnp.float32),
            pltpu.VMEM((2, L), jnp.float32),
            pltpu.VMEM((2 * NS, L), jnp.float32),
            pltpu.VMEM((L,), jnp.float32),
            pltpu.VMEM_SHARED((2 * NS, L), jnp.float32),
            pltpu.SemaphoreType.DMA,
            pltpu.SemaphoreType.DMA,
            pltpu.SemaphoreType.DMA,
        ],
    )(_body)
    return run(tab, packed)[0]
